# baseline (speedup 1.0000x reference)
import functools

import jax
import jax.numpy as jnp
from jax import lax
from jax.experimental import pallas as pl
from jax.experimental.pallas import tpu as pltpu
from jax.experimental.pallas import tpu_sc as plsc
import numpy as np

N = 10000
E = 160000
HID = 512
DEPTH = 4
K = 2
NSTEPS = 100
NT = 25

BN = 1000
GRID = N // BN
NSLICE = 8
FSL = HID // NSLICE

SC_TILES = 16
CHUNK = 128
EPC_UNIT = SC_TILES * CHUNK * 2
E_PAD = ((E + EPC_UNIT - 1) // EPC_UNIT) * EPC_UNIT
EPT = E_PAD // SC_TILES
NCHUNK = EPT // CHUNK
NPAD = 10240
ROWS_PT = NPAD // SC_TILES


def _precompute_body(wt1_ref, bt1_ref, wt2_ref, bt2_ref, wc_ref, bc_ref,
                     table_ref, const_ref):
    half = HID // 2
    tvals = lax.broadcasted_iota(jnp.int32, (128, half), 0).astype(jnp.float32)
    freqs = jnp.exp(
        (-np.log(10000.0) / half)
        * lax.broadcasted_iota(jnp.int32, (128, half), 1).astype(jnp.float32))
    args = tvals * freqs
    temb = jnp.concatenate([jnp.sin(args), jnp.cos(args)], axis=-1)
    z = jnp.dot(temb, wt1_ref[...], preferred_element_type=jnp.float32, precision=lax.Precision.HIGHEST) + bt1_ref[...]
    z = z * jax.nn.sigmoid(z)
    table_ref[...] = (jnp.dot(z, wt2_ref[...], preferred_element_type=jnp.float32, precision=lax.Precision.HIGHEST)
                      + bt2_ref[...])

    t_x = lax.broadcasted_iota(jnp.int32, (1, NT), 1).astype(jnp.float32)
    s1 = t_x * (2.0 * np.pi / NT)
    s2 = t_x * (2.0 * np.pi / (NT // 4 + 1))
    emb = jnp.concatenate([jnp.sin(s1), jnp.cos(s1), jnp.sin(s2), jnp.cos(s2)],
                          axis=1)
    const_ref[...] = (jnp.dot(emb, wc_ref[...], preferred_element_type=jnp.float32, precision=lax.Precision.HIGHEST)
                      + bc_ref[...])


def _precompute(W_t1, b_t1, W_t2, b_t2, W_cond_emb, b_cond):
    return pl.pallas_call(
        _precompute_body,
        out_shape=(jax.ShapeDtypeStruct((128, HID), jnp.float32),
                   jax.ShapeDtypeStruct((1, HID), jnp.float32)),
    )(W_t1, b_t1[None, :], W_t2, b_t2[None, :], W_cond_emb, b_cond[None, :])


def _stage_pre_body(yt_ref, t_ref, x_ref, win_ref, wc_ref, table_ref, const_ref,
                    h_ref):
    onehot = (t_ref[...] == lax.broadcasted_iota(jnp.int32, (1, 128), 1)
              ).astype(jnp.float32)
    h = yt_ref[...] * win_ref[...]
    h = h + jnp.dot(onehot, table_ref[...], preferred_element_type=jnp.float32, precision=lax.Precision.HIGHEST)
    h = h + jnp.dot(x_ref[...], wc_ref[...], preferred_element_type=jnp.float32, precision=lax.Precision.HIGHEST)
    h_ref[...] = h + const_ref[...]


def _stage_pre(y_t, t2, x200, W_in, W_cond_x, table, const):
    return pl.pallas_call(
        _stage_pre_body,
        grid=(GRID,),
        in_specs=[
            pl.BlockSpec((BN, 1), lambda i: (i, 0)),
            pl.BlockSpec((BN, 1), lambda i: (i, 0)),
            pl.BlockSpec((BN, 200), lambda i: (i, 0)),
            pl.BlockSpec((1, HID), lambda i: (0, 0)),
            pl.BlockSpec((200, HID), lambda i: (0, 0)),
            pl.BlockSpec((128, HID), lambda i: (0, 0)),
            pl.BlockSpec((1, HID), lambda i: (0, 0)),
        ],
        out_specs=pl.BlockSpec((BN, HID), lambda i: (i, 0)),
        out_shape=jax.ShapeDtypeStruct((N, HID), jnp.float32),
    )(y_t, t2, x200, W_in, W_cond_x, table, const)


def _stats_body(h_ref, s_ref, o_ref):
    @pl.when(pl.program_id(0) == 0)
    def _():
        o_ref[...] = jnp.zeros_like(o_ref)
    hsh = h_ref[...] - s_ref[...]
    s = jnp.sum(hsh, axis=0, keepdims=True)
    q = jnp.sum(hsh * hsh, axis=0, keepdims=True)
    o_ref[...] += jnp.concatenate([s, q], axis=0)


def _stats(h):
    return pl.pallas_call(
        _stats_body,
        grid=(GRID,),
        in_specs=[pl.BlockSpec((BN, HID), lambda i: (i, 0)),
                  pl.BlockSpec((1, HID), lambda i: (0, 0))],
        out_specs=pl.BlockSpec((2, HID), lambda i: (0, 0)),
        out_shape=jax.ShapeDtypeStruct((2, HID), jnp.float32),
        compiler_params=pltpu.CompilerParams(
            dimension_semantics=("arbitrary",)),
    )(h, lax.slice(h, (0, 0), (1, HID)))


def _bn_relu_w0_body(h_ref, st_ref, s_ref, g_ref, b_ref, w0_ref, ha_ref,
                     o0_ref):
    st = st_ref[...]
    dmu = st[0:1] * (1.0 / N)
    mu = s_ref[...] + dmu
    var = st[1:2] * (1.0 / N) - dmu * dmu
    ha = jnp.maximum(
        (h_ref[...] - mu) / jnp.sqrt(var + 1e-5) * g_ref[...] + b_ref[...],
        0.0)
    for j in range(NSLICE):
        ha_ref[j] = ha[:, j * FSL:(j + 1) * FSL]
    o0_ref[...] = jnp.dot(ha, w0_ref[...], preferred_element_type=jnp.float32, precision=lax.Precision.HIGHEST)


def _bn_relu_w0(h, stats, g, b, W0):
    return pl.pallas_call(
        _bn_relu_w0_body,
        grid=(GRID,),
        in_specs=[
            pl.BlockSpec((BN, HID), lambda i: (i, 0)),
            pl.BlockSpec((2, HID), lambda i: (0, 0)),
            pl.BlockSpec((1, HID), lambda i: (0, 0)),
            pl.BlockSpec((1, HID), lambda i: (0, 0)),
            pl.BlockSpec((1, HID), lambda i: (0, 0)),
            pl.BlockSpec((HID, HID), lambda i: (0, 0)),
        ],
        out_specs=(pl.BlockSpec((NSLICE, BN, FSL), lambda i: (0, i, 0)),
                   pl.BlockSpec((BN, HID), lambda i: (i, 0))),
        out_shape=(jax.ShapeDtypeStruct((NSLICE, NPAD, FSL), jnp.float32),
                   jax.ShapeDtypeStruct((N, HID), jnp.float32)),
    )(h, stats, lax.slice(h, (0, 0), (1, HID)), g, b, W0)


def _mm_add_body(m_ref, p_ref, w_ref, o_ref):
    m = jnp.concatenate([m_ref[j] for j in range(NSLICE)], axis=1)
    o_ref[...] = p_ref[...] + jnp.dot(m, w_ref[...],
                                      preferred_element_type=jnp.float32, precision=lax.Precision.HIGHEST)


def _mm_add(m_sl, p, W):
    return pl.pallas_call(
        _mm_add_body,
        grid=(GRID,),
        in_specs=[
            pl.BlockSpec((NSLICE, BN, FSL), lambda i: (0, i, 0)),
            pl.BlockSpec((BN, HID), lambda i: (i, 0)),
            pl.BlockSpec((HID, HID), lambda i: (0, 0)),
        ],
        out_specs=pl.BlockSpec((BN, HID), lambda i: (i, 0)),
        out_shape=jax.ShapeDtypeStruct((N, HID), jnp.float32),
    )(m_sl, p, W)


def _final_body(m_ref, h_ref, p_ref, w_ref, b_ref, o_ref):
    m = jnp.concatenate([m_ref[j] for j in range(NSLICE)], axis=1)
    out = (p_ref[...] + jnp.dot(m, w_ref[...],
                                preferred_element_type=jnp.float32,
                                precision=lax.Precision.HIGHEST)) + b_ref[...]
    o_ref[...] = h_ref[...] + out


def _final(m_sl, h, p, W, b):
    return pl.pallas_call(
        _final_body,
        grid=(GRID,),
        in_specs=[
            pl.BlockSpec((NSLICE, BN, FSL), lambda i: (0, i, 0)),
            pl.BlockSpec((BN, HID), lambda i: (i, 0)),
            pl.BlockSpec((BN, HID), lambda i: (i, 0)),
            pl.BlockSpec((HID, HID), lambda i: (0, 0)),
            pl.BlockSpec((1, HID), lambda i: (0, 0)),
        ],
        out_specs=pl.BlockSpec((BN, HID), lambda i: (i, 0)),
        out_shape=jax.ShapeDtypeStruct((N, HID), jnp.float32),
    )(m_sl, h, p, W, b)


def _out_proj_body(h_ref, w_ref, b_ref, o_ref):
    o_ref[...] = jnp.dot(h_ref[...], w_ref[...],
                         preferred_element_type=jnp.float32, precision=lax.Precision.HIGHEST) + b_ref[...]


def _out_proj(h, W_out, b_out):
    return pl.pallas_call(
        _out_proj_body,
        grid=(GRID,),
        in_specs=[
            pl.BlockSpec((BN, HID), lambda i: (i, 0)),
            pl.BlockSpec((HID, 1), lambda i: (0, 0)),
            pl.BlockSpec((1, 1), lambda i: (0, 0)),
        ],
        out_specs=pl.BlockSpec((BN, 1), lambda i: (i, 0)),
        out_shape=jax.ShapeDtypeStruct((N, 1), jnp.float32),
    )(h, W_out, b_out)


_DNUMS = lax.GatherDimensionNumbers(
    offset_dims=(), collapsed_slice_dims=(0,), start_index_map=(0,))


def _scale_chunk(rows_b, wv, g):
    def grp_body(gi, _):
        wreg = wv[g, pl.ds(gi * 16, 16)]
        for j in range(16):
            wspl = lax.gather(
                wreg, jnp.full((16, 1), j, jnp.int32), _DNUMS,
                slice_sizes=(1,),
                mode=lax.GatherScatterMode.PROMISE_IN_BOUNDS)
            e = gi * 16 + j
            for f in range(FSL // 16):
                rows_b[e, pl.ds(f * 16, 16)] = (
                    rows_b[e, pl.ds(f * 16, 16)] * wspl)
        return 0

    lax.fori_loop(0, CHUNK // 16, grp_body, 0)


def _propagate_body(h_hbm, src_hbm, dst_hbm, w_hbm, out_hbm,
                    srcv, dstv, wv, rows0, rows1, zbuf, acc, semg, semsc):
    c = lax.axis_index("c")
    s = lax.axis_index("s")
    rows = (rows0, rows1)

    pltpu.sync_copy(src_hbm.at[pl.ds(s * NCHUNK, NCHUNK)], dstv)
    pltpu.sync_copy(w_hbm.at[pl.ds(s * NCHUNK, NCHUNK)], wv)

    z16 = jnp.zeros((16,), jnp.float32)

    def zb_body(r, _):
        for j in range(FSL // 16):
            zbuf[r, pl.ds(j * 16, 16)] = z16
        return 0

    lax.fori_loop(0, CHUNK, zb_body, 0)

    my0 = s * ROWS_PT
    for js in range(NSLICE // 2):
        fs = c * (NSLICE // 2) + js
        row0 = fs * NPAD

        if js == 0:
            def sh_body(r, _):
                for j in range(CHUNK // 16):
                    srcv[r, pl.ds(j * 16, 16)] = (
                        dstv[r, pl.ds(j * 16, 16)] + row0)
                return 0
        else:
            def sh_body(r, _):
                for j in range(CHUNK // 16):
                    srcv[r, pl.ds(j * 16, 16)] = (
                        srcv[r, pl.ds(j * 16, 16)] + NPAD)
                return 0
        lax.fori_loop(0, NCHUNK, sh_body, 0)

        if js == 0:
            pltpu.sync_copy(dst_hbm.at[pl.ds(s * NCHUNK, NCHUNK)], dstv)

        for z in range(ROWS_PT // CHUNK):
            pltpu.sync_copy(zbuf.at[...], acc.at[pl.ds(my0 + z * CHUNK, CHUNK)])
        plsc.subcore_barrier()

        pltpu.async_copy(h_hbm.at[srcv.at[0]], rows0, semg)

        def pair_body(g2, _):
            for b in range(2):
                g = g2 * 2 + b
                rb = rows[b]
                ro = rows[1 - b]

                @pl.when(g >= 1)
                def _():
                    pltpu.make_async_copy(ro, acc.at[dstv.at[g]], semsc).wait()

                @pl.when(g + 1 < NCHUNK)
                def _():
                    pltpu.async_copy(h_hbm.at[srcv.at[g + 1]], ro, semg)

                pltpu.make_async_copy(h_hbm.at[srcv.at[g]], rb, semg).wait()
                _scale_chunk(rb, wv, g)
                pltpu.async_copy(rb, acc.at[dstv.at[g]], semsc, add=True)
            return 0

        lax.fori_loop(0, NCHUNK // 2, pair_body, 0)
        pltpu.make_async_copy(rows1, acc.at[dstv.at[NCHUNK - 1]], semsc).wait()
        plsc.subcore_barrier()

        for z in range(ROWS_PT // CHUNK):
            pltpu.sync_copy(acc.at[pl.ds(my0 + z * CHUNK, CHUNK)],
                            out_hbm.at[pl.ds(row0 + my0 + z * CHUNK, CHUNK)])


@jax.jit
def _propagate(h_sl, src, dst, w):
    mesh = plsc.VectorSubcoreMesh(core_axis_name="c", subcore_axis_name="s")
    return pl.kernel(
        _propagate_body,
        out_type=jax.ShapeDtypeStruct((NSLICE * NPAD, FSL), jnp.float32),
        mesh=mesh,
        compiler_params=pltpu.CompilerParams(use_tc_tiling_on_sc=False),
        scratch_types=[
            pltpu.VMEM((NCHUNK, CHUNK), jnp.int32),
            pltpu.VMEM((NCHUNK, CHUNK), jnp.int32),
            pltpu.VMEM((NCHUNK, CHUNK), jnp.float32),
            pltpu.VMEM((CHUNK, FSL), jnp.float32),
            pltpu.VMEM((CHUNK, FSL), jnp.float32),
            pltpu.VMEM((CHUNK, FSL), jnp.float32),
            pltpu.VMEM_SHARED((NPAD, FSL), jnp.float32),
            pltpu.SemaphoreType.DMA,
            pltpu.SemaphoreType.DMA,
        ],
    )(h_sl, src, dst, w)


def kernel(y_t, t, x, edge_index, edge_weight, W_in, b_in, W_cond, b_cond,
           W_t1, b_t1, W_t2, b_t2, W_conv, b_conv, gamma, beta, W_out, b_out):
    x200 = x.reshape(N, -1)
    t2 = t.astype(jnp.int32).reshape(N, 1)
    src = edge_index[0].astype(jnp.int32)
    dst = edge_index[1].astype(jnp.int32)
    pad = E_PAD - E
    spread = (jnp.arange(pad, dtype=jnp.int32) * 64) % N
    src = jnp.concatenate([src, spread]).reshape(E_PAD // CHUNK, CHUNK)
    dst = jnp.concatenate([dst, spread]).reshape(E_PAD // CHUNK, CHUNK)
    w = jnp.concatenate([edge_weight.astype(jnp.float32),
                         jnp.zeros((pad,), jnp.float32)]
                        ).reshape(E_PAD // CHUNK, CHUNK)

    table, const = _precompute(W_t1, b_t1, W_t2, b_t2, W_cond[200:], b_cond)
    const = const + b_in[None, :]
    h = _stage_pre(y_t, t2, x200, W_in, W_cond[:200], table, const)

    for i in range(DEPTH):
        st = _stats(h)
        ha_sl, p0 = _bn_relu_w0(h, st, gamma[i][None, :], beta[i][None, :],
                                W_conv[i, 0])
        m1 = _propagate(ha_sl.reshape(NSLICE * NPAD, FSL), src, dst, w)
        m1_sl = m1.reshape(NSLICE, NPAD, FSL)
        p1 = _mm_add(m1_sl, p0, W_conv[i, 1])
        m2 = _propagate(m1, src, dst, w)
        h = _final(m2.reshape(NSLICE, NPAD, FSL), h, p1, W_conv[i, 2],
                   b_conv[i][None, :])

    return _out_proj(h, W_out, b_out[None, :])


# speedup vs baseline: 2.1691x; 2.1691x over previous
import functools

import jax
import jax.numpy as jnp
from jax import lax
from jax.experimental import pallas as pl
from jax.experimental.pallas import tpu as pltpu
from jax.experimental.pallas import tpu_sc as plsc
import numpy as np

N = 10000
E = 160000
HID = 512
DEPTH = 4
K = 2
NSTEPS = 100
NT = 25

BN = 1000
GRID = N // BN
NSLICE = 8
FSL = HID // NSLICE

SC_TILES = 16
CHUNK = 128
EPC_UNIT = SC_TILES * CHUNK * 2
E_PAD = ((E + EPC_UNIT - 1) // EPC_UNIT) * EPC_UNIT
EPT = E_PAD // SC_TILES
NCHUNK = EPT // CHUNK
NPAD = 10240
ROWS_PT = NPAD // SC_TILES


def _precompute_body(wt1_ref, bt1_ref, wt2_ref, bt2_ref, wc_ref, bc_ref,
                     table_ref, const_ref):
    half = HID // 2
    tvals = lax.broadcasted_iota(jnp.int32, (128, half), 0).astype(jnp.float32)
    freqs = jnp.exp(
        (-np.log(10000.0) / half)
        * lax.broadcasted_iota(jnp.int32, (128, half), 1).astype(jnp.float32))
    args = tvals * freqs
    temb = jnp.concatenate([jnp.sin(args), jnp.cos(args)], axis=-1)
    z = jnp.dot(temb, wt1_ref[...], preferred_element_type=jnp.float32, precision=lax.Precision.HIGHEST) + bt1_ref[...]
    z = z * jax.nn.sigmoid(z)
    table_ref[...] = (jnp.dot(z, wt2_ref[...], preferred_element_type=jnp.float32, precision=lax.Precision.HIGHEST)
                      + bt2_ref[...])

    t_x = lax.broadcasted_iota(jnp.int32, (1, NT), 1).astype(jnp.float32)
    s1 = t_x * (2.0 * np.pi / NT)
    s2 = t_x * (2.0 * np.pi / (NT // 4 + 1))
    emb = jnp.concatenate([jnp.sin(s1), jnp.cos(s1), jnp.sin(s2), jnp.cos(s2)],
                          axis=1)
    const_ref[...] = (jnp.dot(emb, wc_ref[...], preferred_element_type=jnp.float32, precision=lax.Precision.HIGHEST)
                      + bc_ref[...])


def _precompute(W_t1, b_t1, W_t2, b_t2, W_cond_emb, b_cond):
    return pl.pallas_call(
        _precompute_body,
        out_shape=(jax.ShapeDtypeStruct((128, HID), jnp.float32),
                   jax.ShapeDtypeStruct((1, HID), jnp.float32)),
    )(W_t1, b_t1[None, :], W_t2, b_t2[None, :], W_cond_emb, b_cond[None, :])


def _stage_pre_body(yt_ref, t_ref, x_ref, win_ref, wc_ref, table_ref, const_ref,
                    h_ref):
    onehot = (t_ref[...] == lax.broadcasted_iota(jnp.int32, (1, 128), 1)
              ).astype(jnp.float32)
    h = yt_ref[...] * win_ref[...]
    h = h + jnp.dot(onehot, table_ref[...], preferred_element_type=jnp.float32, precision=lax.Precision.HIGHEST)
    h = h + jnp.dot(x_ref[...], wc_ref[...], preferred_element_type=jnp.float32, precision=lax.Precision.HIGHEST)
    h_ref[...] = h + const_ref[...]


def _stage_pre(y_t, t2, x200, W_in, W_cond_x, table, const):
    return pl.pallas_call(
        _stage_pre_body,
        grid=(GRID,),
        in_specs=[
            pl.BlockSpec((BN, 1), lambda i: (i, 0)),
            pl.BlockSpec((BN, 1), lambda i: (i, 0)),
            pl.BlockSpec((BN, 200), lambda i: (i, 0)),
            pl.BlockSpec((1, HID), lambda i: (0, 0)),
            pl.BlockSpec((200, HID), lambda i: (0, 0)),
            pl.BlockSpec((128, HID), lambda i: (0, 0)),
            pl.BlockSpec((1, HID), lambda i: (0, 0)),
        ],
        out_specs=pl.BlockSpec((BN, HID), lambda i: (i, 0)),
        out_shape=jax.ShapeDtypeStruct((N, HID), jnp.float32),
    )(y_t, t2, x200, W_in, W_cond_x, table, const)


def _stats_body(h_ref, s_ref, o_ref):
    @pl.when(pl.program_id(0) == 0)
    def _():
        o_ref[...] = jnp.zeros_like(o_ref)
    hsh = h_ref[...] - s_ref[...]
    s = jnp.sum(hsh, axis=0, keepdims=True)
    q = jnp.sum(hsh * hsh, axis=0, keepdims=True)
    o_ref[...] += jnp.concatenate([s, q], axis=0)


def _stats(h):
    return pl.pallas_call(
        _stats_body,
        grid=(GRID,),
        in_specs=[pl.BlockSpec((BN, HID), lambda i: (i, 0)),
                  pl.BlockSpec((1, HID), lambda i: (0, 0))],
        out_specs=pl.BlockSpec((2, HID), lambda i: (0, 0)),
        out_shape=jax.ShapeDtypeStruct((2, HID), jnp.float32),
        compiler_params=pltpu.CompilerParams(
            dimension_semantics=("arbitrary",)),
    )(h, lax.slice(h, (0, 0), (1, HID)))


def _bn_relu_w0_body(h_ref, st_ref, s_ref, g_ref, b_ref, w0_ref, ha_ref,
                     o0_ref):
    st = st_ref[...]
    dmu = st[0:1] * (1.0 / N)
    mu = s_ref[...] + dmu
    var = st[1:2] * (1.0 / N) - dmu * dmu
    ha = jnp.maximum(
        (h_ref[...] - mu) / jnp.sqrt(var + 1e-5) * g_ref[...] + b_ref[...],
        0.0)
    for j in range(NSLICE):
        ha_ref[j] = ha[:, j * FSL:(j + 1) * FSL]
    o0_ref[...] = jnp.dot(ha, w0_ref[...], preferred_element_type=jnp.float32, precision=lax.Precision.HIGHEST)


def _bn_relu_w0(h, stats, g, b, W0):
    return pl.pallas_call(
        _bn_relu_w0_body,
        grid=(GRID,),
        in_specs=[
            pl.BlockSpec((BN, HID), lambda i: (i, 0)),
            pl.BlockSpec((2, HID), lambda i: (0, 0)),
            pl.BlockSpec((1, HID), lambda i: (0, 0)),
            pl.BlockSpec((1, HID), lambda i: (0, 0)),
            pl.BlockSpec((1, HID), lambda i: (0, 0)),
            pl.BlockSpec((HID, HID), lambda i: (0, 0)),
        ],
        out_specs=(pl.BlockSpec((NSLICE, BN, FSL), lambda i: (0, i, 0)),
                   pl.BlockSpec((BN, HID), lambda i: (i, 0))),
        out_shape=(jax.ShapeDtypeStruct((NSLICE, NPAD, FSL), jnp.float32),
                   jax.ShapeDtypeStruct((N, HID), jnp.float32)),
    )(h, stats, lax.slice(h, (0, 0), (1, HID)), g, b, W0)


def _mm_add_body(m_ref, p_ref, w_ref, o_ref):
    m = jnp.concatenate([m_ref[j] for j in range(NSLICE)], axis=1)
    o_ref[...] = p_ref[...] + jnp.dot(m, w_ref[...],
                                      preferred_element_type=jnp.float32, precision=lax.Precision.HIGHEST)


def _mm_add(m_sl, p, W):
    return pl.pallas_call(
        _mm_add_body,
        grid=(GRID,),
        in_specs=[
            pl.BlockSpec((NSLICE, BN, FSL), lambda i: (0, i, 0)),
            pl.BlockSpec((BN, HID), lambda i: (i, 0)),
            pl.BlockSpec((HID, HID), lambda i: (0, 0)),
        ],
        out_specs=pl.BlockSpec((BN, HID), lambda i: (i, 0)),
        out_shape=jax.ShapeDtypeStruct((N, HID), jnp.float32),
    )(m_sl, p, W)


def _final_body(m_ref, h_ref, p_ref, w_ref, b_ref, o_ref):
    m = jnp.concatenate([m_ref[j] for j in range(NSLICE)], axis=1)
    out = (p_ref[...] + jnp.dot(m, w_ref[...],
                                preferred_element_type=jnp.float32,
                                precision=lax.Precision.HIGHEST)) + b_ref[...]
    o_ref[...] = h_ref[...] + out


def _final(m_sl, h, p, W, b):
    return pl.pallas_call(
        _final_body,
        grid=(GRID,),
        in_specs=[
            pl.BlockSpec((NSLICE, BN, FSL), lambda i: (0, i, 0)),
            pl.BlockSpec((BN, HID), lambda i: (i, 0)),
            pl.BlockSpec((BN, HID), lambda i: (i, 0)),
            pl.BlockSpec((HID, HID), lambda i: (0, 0)),
            pl.BlockSpec((1, HID), lambda i: (0, 0)),
        ],
        out_specs=pl.BlockSpec((BN, HID), lambda i: (i, 0)),
        out_shape=jax.ShapeDtypeStruct((N, HID), jnp.float32),
    )(m_sl, h, p, W, b)


def _out_proj_body(h_ref, w_ref, b_ref, o_ref):
    o_ref[...] = jnp.dot(h_ref[...], w_ref[...],
                         preferred_element_type=jnp.float32, precision=lax.Precision.HIGHEST) + b_ref[...]


def _out_proj(h, W_out, b_out):
    return pl.pallas_call(
        _out_proj_body,
        grid=(GRID,),
        in_specs=[
            pl.BlockSpec((BN, HID), lambda i: (i, 0)),
            pl.BlockSpec((HID, 1), lambda i: (0, 0)),
            pl.BlockSpec((1, 1), lambda i: (0, 0)),
        ],
        out_specs=pl.BlockSpec((BN, 1), lambda i: (i, 0)),
        out_shape=jax.ShapeDtypeStruct((N, 1), jnp.float32),
    )(h, W_out, b_out)


_DNUMS = lax.GatherDimensionNumbers(
    offset_dims=(), collapsed_slice_dims=(0,), start_index_map=(0,))


def _scale_chunk(rows_b, wv, g):
    for gi in range(CHUNK // 16):
        wreg = wv[g, pl.ds(gi * 16, 16)]
        for j in range(16):
            wspl = lax.gather(
                wreg, jnp.full((16, 1), j, jnp.int32), _DNUMS,
                slice_sizes=(1,),
                mode=lax.GatherScatterMode.PROMISE_IN_BOUNDS)
            e = gi * 16 + j
            for f in range(FSL // 16):
                rows_b[e, pl.ds(f * 16, 16)] = (
                    rows_b[e, pl.ds(f * 16, 16)] * wspl)


def _propagate_body(h_hbm, src_hbm, dst_hbm, w_hbm, out_hbm,
                    srcv, dstv, wv, rows0, rows1, zbuf, acc, semg, semsc):
    c = lax.axis_index("c")
    s = lax.axis_index("s")
    rows = (rows0, rows1)

    pltpu.sync_copy(src_hbm.at[pl.ds(s * NCHUNK, NCHUNK)], dstv)
    pltpu.sync_copy(w_hbm.at[pl.ds(s * NCHUNK, NCHUNK)], wv)

    z16 = jnp.zeros((16,), jnp.float32)

    def zb_body(r, _):
        for j in range(FSL // 16):
            zbuf[r, pl.ds(j * 16, 16)] = z16
        return 0

    lax.fori_loop(0, CHUNK, zb_body, 0)

    my0 = s * ROWS_PT
    for js in range(NSLICE // 2):
        fs = c * (NSLICE // 2) + js
        row0 = fs * NPAD

        if js == 0:
            def sh_body(r, _):
                for j in range(CHUNK // 16):
                    srcv[r, pl.ds(j * 16, 16)] = (
                        dstv[r, pl.ds(j * 16, 16)] + row0)
                return 0
        else:
            def sh_body(r, _):
                for j in range(CHUNK // 16):
                    srcv[r, pl.ds(j * 16, 16)] = (
                        srcv[r, pl.ds(j * 16, 16)] + NPAD)
                return 0
        lax.fori_loop(0, NCHUNK, sh_body, 0)

        if js == 0:
            pltpu.sync_copy(dst_hbm.at[pl.ds(s * NCHUNK, NCHUNK)], dstv)

        for z in range(ROWS_PT // CHUNK):
            pltpu.sync_copy(zbuf.at[...], acc.at[pl.ds(my0 + z * CHUNK, CHUNK)])
        plsc.subcore_barrier()

        pltpu.async_copy(h_hbm.at[srcv.at[0]], rows0, semg)

        def pair_body(g2, _):
            for b in range(2):
                g = g2 * 2 + b
                rb = rows[b]
                ro = rows[1 - b]

                @pl.when(g >= 1)
                def _():
                    pltpu.make_async_copy(ro, acc.at[dstv.at[g]], semsc).wait()

                @pl.when(g + 1 < NCHUNK)
                def _():
                    pltpu.async_copy(h_hbm.at[srcv.at[g + 1]], ro, semg)

                pltpu.make_async_copy(h_hbm.at[srcv.at[g]], rb, semg).wait()
                _scale_chunk(rb, wv, g)
                pltpu.async_copy(rb, acc.at[dstv.at[g]], semsc, add=True)
            return 0

        lax.fori_loop(0, NCHUNK // 2, pair_body, 0)
        pltpu.make_async_copy(rows1, acc.at[dstv.at[NCHUNK - 1]], semsc).wait()
        plsc.subcore_barrier()

        for z in range(ROWS_PT // CHUNK):
            pltpu.sync_copy(acc.at[pl.ds(my0 + z * CHUNK, CHUNK)],
                            out_hbm.at[pl.ds(row0 + my0 + z * CHUNK, CHUNK)])


@jax.jit
def _propagate(h_sl, src, dst, w):
    mesh = plsc.VectorSubcoreMesh(core_axis_name="c", subcore_axis_name="s")
    return pl.kernel(
        _propagate_body,
        out_type=jax.ShapeDtypeStruct((NSLICE * NPAD, FSL), jnp.float32),
        mesh=mesh,
        compiler_params=pltpu.CompilerParams(use_tc_tiling_on_sc=False),
        scratch_types=[
            pltpu.VMEM((NCHUNK, CHUNK), jnp.int32),
            pltpu.VMEM((NCHUNK, CHUNK), jnp.int32),
            pltpu.VMEM((NCHUNK, CHUNK), jnp.float32),
            pltpu.VMEM((CHUNK, FSL), jnp.float32),
            pltpu.VMEM((CHUNK, FSL), jnp.float32),
            pltpu.VMEM((CHUNK, FSL), jnp.float32),
            pltpu.VMEM_SHARED((NPAD, FSL), jnp.float32),
            pltpu.SemaphoreType.DMA,
            pltpu.SemaphoreType.DMA,
        ],
    )(h_sl, src, dst, w)


def kernel(y_t, t, x, edge_index, edge_weight, W_in, b_in, W_cond, b_cond,
           W_t1, b_t1, W_t2, b_t2, W_conv, b_conv, gamma, beta, W_out, b_out):
    x200 = x.reshape(N, -1)
    t2 = t.astype(jnp.int32).reshape(N, 1)
    src = edge_index[0].astype(jnp.int32)
    dst = edge_index[1].astype(jnp.int32)
    pad = E_PAD - E
    spread = (jnp.arange(pad, dtype=jnp.int32) * 64) % N
    src = jnp.concatenate([src, spread]).reshape(E_PAD // CHUNK, CHUNK)
    dst = jnp.concatenate([dst, spread]).reshape(E_PAD // CHUNK, CHUNK)
    w = jnp.concatenate([edge_weight.astype(jnp.float32),
                         jnp.zeros((pad,), jnp.float32)]
                        ).reshape(E_PAD // CHUNK, CHUNK)

    table, const = _precompute(W_t1, b_t1, W_t2, b_t2, W_cond[200:], b_cond)
    const = const + b_in[None, :]
    h = _stage_pre(y_t, t2, x200, W_in, W_cond[:200], table, const)

    for i in range(DEPTH):
        st = _stats(h)
        ha_sl, p0 = _bn_relu_w0(h, st, gamma[i][None, :], beta[i][None, :],
                                W_conv[i, 0])
        m1 = _propagate(ha_sl.reshape(NSLICE * NPAD, FSL), src, dst, w)
        m1_sl = m1.reshape(NSLICE, NPAD, FSL)
        p1 = _mm_add(m1_sl, p0, W_conv[i, 1])
        m2 = _propagate(m1, src, dst, w)
        h = _final(m2.reshape(NSLICE, NPAD, FSL), h, p1, W_conv[i, 2],
                   b_conv[i][None, :])

    return _out_proj(h, W_out, b_out[None, :])
